# baseline (speedup 1.0000x reference)
import functools

import jax
import jax.numpy as jnp
from jax import lax
from jax.experimental import pallas as pl
from jax.experimental.pallas import tpu as pltpu
from jax.experimental.pallas import tpu_sc as plsc

B, L = 4096, 200
NTOK = B * L
INTD = 21
D = 32
HID = 64
EPS = 1e-6

NC, NS = 2, 16
NW = NC * NS
TOK_PER_W = NTOK // NW
CH = 256
CHB = CH // 128
NCHUNK = TOK_PER_W // CH


def _proj_body(e1, e2, e3, e4, w1, w2, w3, w4, p1, p2, p3, p4):
    p1[...] = jnp.dot(e1[...], w1[...], preferred_element_type=jnp.float32)
    p2[...] = jnp.dot(e2[...], w2[...], preferred_element_type=jnp.float32)
    p3[...] = jnp.dot(e3[...], w3[...], preferred_element_type=jnp.float32)
    p4[...] = jnp.dot(e4[...], w4[...], preferred_element_type=jnp.float32)


def _project_tables(E1, E2, E3, E4, W_cat):
    w1, w2, w3, w4 = (W_cat[0:21], W_cat[21:42], W_cat[42:63], W_cat[63:84])
    out_shapes = tuple(
        jax.ShapeDtypeStruct((e.shape[0], D), jnp.float32)
        for e in (E1, E2, E3, E4)
    )
    return pl.pallas_call(
        _proj_body,
        out_shape=out_shapes,
    )(E1, E2, E3, E4, w1, w2, w3, w4)


def _sc_body(ix, p1, p2, p3, p4, out,
             idxv, r1, r2, r3, r4, sv, semi, semg0, semg1, semo0, semo1):
    wid = lax.axis_index("s") * NC + lax.axis_index("c")
    tabs = (p1, p2, p3, p4)
    rbufs = (r1, r2, r3, r4)

    def brow(c):
        return (wid * TOK_PER_W + c * CH) // 128

    def base(c):
        return wid * TOK_PER_W + c * CH

    def fire_idx(c, b):
        pltpu.async_copy(ix.at[pl.ds(brow(c), CHB)], idxv.at[b], semi)

    def wait_idx(b):
        pltpu.make_async_copy(ix.at[pl.ds(0, CHB)], idxv.at[b], semi).wait()

    def fire_gathers(c, b, semg):
        for t in range(4):
            for j in range(CHB):
                pltpu.async_copy(
                    tabs[t].at[idxv.at[b, j, t]],
                    rbufs[t].at[b].at[pl.ds(j * 128, 128)],
                    semg)

    def wait_gathers(b, semg):
        for t in range(4):
            for j in range(CHB):
                pltpu.make_async_copy(
                    p1.at[pl.ds(0, 128)],
                    rbufs[t].at[b].at[pl.ds(j * 128, 128)],
                    semg).wait()

    def fire_out(c, b, semo):
        pltpu.async_copy(sv.at[b], out.at[pl.ds(base(c), CH)], semo)

    def wait_out(c, b, semo):
        pltpu.make_async_copy(sv.at[b], out.at[pl.ds(base(c), CH)],
                              semo).wait()

    UNROLL = 8

    def do_sum(b):
        def sum_body(t, carry):
            for u in range(UNROLL):
                row = t * UNROLL + u
                for k in range(2):
                    sl = pl.ds(k * 16, 16)
                    sv[b, row, sl] = (r1[b, row, sl] + r2[b, row, sl]
                                      + r3[b, row, sl] + r4[b, row, sl])
            return carry
        lax.fori_loop(0, CH // UNROLL, sum_body, None)

    def chunk(c, b, semg_b, semg_o, semo_b):
        @pl.when(c >= 2)
        def _():
            wait_out(c - 2, b, semo_b)

        @pl.when(c + 1 < NCHUNK)
        def _():
            wait_idx(1 - b)
            fire_gathers(c + 1, 1 - b, semg_o)

        wait_gathers(b, semg_b)

        @pl.when(c + 2 < NCHUNK)
        def _():
            fire_idx(c + 2, b)

        do_sum(b)
        fire_out(c, b, semo_b)

    fire_idx(0, 0)
    wait_idx(0)
    fire_gathers(0, 0, semg0)
    fire_idx(1, 1)

    def super_body(i, carry):
        chunk(2 * i, 0, semg0, semg1, semo0)
        chunk(2 * i + 1, 1, semg1, semg0, semo1)
        return carry

    lax.fori_loop(0, NCHUNK // 2, super_body, None)
    wait_out(NCHUNK - 2, 0, semo0)
    wait_out(NCHUNK - 1, 1, semo1)


def _sc_gather_sum(ix, P1, P2, P3, P4):
    mesh = plsc.VectorSubcoreMesh(core_axis_name="c", subcore_axis_name="s")
    fn = functools.partial(
        pl.kernel,
        out_type=jax.ShapeDtypeStruct((NTOK, D), jnp.float32),
        mesh=mesh,
        scratch_types=[
            pltpu.VMEM((2, CHB, 4, 128), jnp.int32),
            pltpu.VMEM((2, CH, D), jnp.float32),
            pltpu.VMEM((2, CH, D), jnp.float32),
            pltpu.VMEM((2, CH, D), jnp.float32),
            pltpu.VMEM((2, CH, D), jnp.float32),
            pltpu.VMEM((2, CH, D), jnp.float32),
            pltpu.SemaphoreType.DMA,
            pltpu.SemaphoreType.DMA,
            pltpu.SemaphoreType.DMA,
            pltpu.SemaphoreType.DMA,
            pltpu.SemaphoreType.DMA,
        ],
        compiler_params=pltpu.CompilerParams(use_tc_tiling_on_sc=False),
    )(_sc_body)
    return fn(ix, P1, P2, P3, P4)


def _ln_body(s_ref, x_ref, bcat, gcat, btcat, wnum, bnum, gnum, btnum,
             gout, btout, o_ref):
    s = s_ref[...] + bcat[...]
    mu = jnp.mean(s, axis=-1, keepdims=True)
    xc = s - mu
    var = jnp.mean(xc * xc, axis=-1, keepdims=True)
    cat = xc * lax.rsqrt(var + EPS) * gcat[...] + btcat[...]

    x = x_ref[...]
    h = x * wnum[...] + bnum[...]
    mu2 = jnp.mean(h, axis=-1, keepdims=True)
    hc = h - mu2
    var2 = jnp.mean(hc * hc, axis=-1, keepdims=True)
    num = hc * lax.rsqrt(var2 + EPS) * gnum[...] + btnum[...]

    o = jnp.concatenate([cat, num], axis=-1)
    mu3 = jnp.mean(o, axis=-1, keepdims=True)
    oc = o - mu3
    var3 = jnp.mean(oc * oc, axis=-1, keepdims=True)
    o_ref[...] = oc * lax.rsqrt(var3 + EPS) * gout[...] + btout[...]


def _ln_stage(s, x, b_cat, g_cat, bt_cat, W_num, b_num, g_num, bt_num,
              g_out, bt_out):
    TBLK = 2048
    grid = (NTOK // TBLK,)
    row = lambda i: (i, 0)
    const = lambda i: (0, 0)
    vec32 = pl.BlockSpec((1, D), const)
    vec64 = pl.BlockSpec((1, HID), const)
    return pl.pallas_call(
        _ln_body,
        grid=grid,
        in_specs=[
            pl.BlockSpec((TBLK, D), row),
            pl.BlockSpec((TBLK, 1), row),
            vec32, vec32, vec32, vec32, vec32, vec32, vec32,
            vec64, vec64,
        ],
        out_specs=pl.BlockSpec((TBLK, HID), row),
        out_shape=jax.ShapeDtypeStruct((NTOK, HID), jnp.float32),
    )(s, x,
      b_cat.reshape(1, D), g_cat.reshape(1, D), bt_cat.reshape(1, D),
      W_num.reshape(1, D), b_num.reshape(1, D), g_num.reshape(1, D),
      bt_num.reshape(1, D), g_out.reshape(1, HID), bt_out.reshape(1, HID))


def kernel(test, question, tag, testTag, num_feat,
           E_test, E_q, E_tag, E_tt,
           W_cat, b_cat, g_cat, bt_cat,
           W_num, b_num, g_num, bt_num,
           g_out, bt_out):
    P1, P2, P3, P4 = _project_tables(E_test, E_q, E_tag, E_tt, W_cat)
    idx = lambda a: a.reshape(NTOK // 128, 128)
    ix = jnp.stack([idx(test), idx(question), idx(tag), idx(testTag)], axis=1)
    s = _sc_gather_sum(ix, P1, P2, P3, P4)
    out = _ln_stage(s, num_feat.reshape(NTOK, 1),
                    b_cat, g_cat, bt_cat, W_num, b_num, g_num, bt_num,
                    g_out, bt_out)
    return out.reshape(B, L, HID)
